# baseline (speedup 1.0000x reference)
import functools

import jax
import jax.numpy as jnp
from jax import lax
from jax.experimental import pallas as pl
from jax.experimental.pallas import tpu as pltpu
from jax.experimental.pallas import tpu_sc as plsc

N = 10000
N_PAD = 10240
E = 320000
EB = 128
E_PAD = 331776
ROW_BLK = 400
N_SLICE = N_PAD // 16
C = 128


SB = 9
SEB = SB * EB


def _agg_body(K, split_edges, src_hbm, dst_hbm, ew_hbm, h_hbm, out_hbm,
              srcbuf, dstbuf, ewbuf, rows_a, rows_b, acc,
              sem_a, sem_b, sem_sa, sem_sb):
    cid = lax.axis_index("c")
    sid = lax.axis_index("s")

    my_slice = pl.ds(sid * N_SLICE, N_SLICE)

    def zero_acc_slice():
        def zrow(r, carry):
            for j in range(C // 16):
                rows_a[r, pl.ds(j * 16, 16)] = jnp.zeros((16,), jnp.float32)
            return carry
        lax.fori_loop(0, EB, zrow, 0)
        for i in range(N_SLICE // EB):
            pltpu.sync_copy(rows_a, acc.at[pl.ds(sid * N_SLICE + i * EB, EB)])

    def edge_loop(koff, e_base, ngrp):
        koffv = jnp.full((16,), koff, jnp.int32)
        bufs = [(rows_a, sem_a, sem_sa), (rows_b, sem_b, sem_sb)]

        def grp_body(g, carry):
            e0 = e_base + g * SEB
            pltpu.sync_copy(src_hbm.at[pl.ds(e0, SEB)], srcbuf)
            pltpu.sync_copy(dst_hbm.at[pl.ds(e0, SEB)], dstbuf.at[0])
            pltpu.sync_copy(ew_hbm.at[pl.ds(e0, SEB)], ewbuf)
            for j in range(SEB // 16):
                sl = pl.ds(j * 16, 16)
                srcbuf[sl] = srcbuf[sl] + koffv

            gh = [None, None]
            sh = [None, None]

            def issue(b):
                r, s, _ = bufs[b % 2]
                if sh[b % 2] is not None:
                    sh[b % 2].wait()
                    sh[b % 2] = None
                gh[b % 2] = pltpu.async_copy(
                    h_hbm.at[srcbuf.at[pl.ds(b * EB, EB)]], r, s)

            def drain(b):
                r, _, ss = bufs[b % 2]
                gh[b % 2].wait()

                def scale_body(e4, c2):
                    for u in range(4):
                        e = e4 * 4 + u
                        ev = plsc.load_gather(
                            ewbuf, [jnp.full((16,), b * EB, jnp.int32) + e])
                        for j in range(C // 16):
                            sl = pl.ds(j * 16, 16)
                            r[e, sl] = r[e, sl] * ev
                    return c2
                lax.fori_loop(0, EB // 4, scale_body, 0)
                sh[b % 2] = pltpu.async_copy(
                    r, acc.at[dstbuf.at[0, pl.ds(b * EB, EB)]], ss, add=True)

            issue(0)
            for b in range(SB):
                if b + 1 < SB:
                    issue(b + 1)
                drain(b)
            for h in sh:
                if h is not None:
                    h.wait()
            return carry
        lax.fori_loop(0, ngrp, grp_body, 0)

    if split_edges:
        wid = cid * 16 + sid
        per_tile = E_PAD // 32
        zero_acc_slice()
        plsc.subcore_barrier()
        edge_loop(jnp.int32(0), wid * per_tile, per_tile // SEB)
        plsc.subcore_barrier()
        pltpu.sync_copy(acc.at[my_slice], out_hbm.at[cid, my_slice])
    else:
        per_tile = E_PAD // 16
        for k_local in range(K // 2):
            k = cid * (K // 2) + k_local
            zero_acc_slice()
            plsc.subcore_barrier()
            edge_loop(k * N, sid * per_tile, per_tile // SEB)
            plsc.subcore_barrier()
            pltpu.sync_copy(acc.at[my_slice], out_hbm.at[k, my_slice])


def _make_agg(K, split_edges):
    n_out = 2 if split_edges else K
    mesh = plsc.VectorSubcoreMesh(core_axis_name="c", subcore_axis_name="s")
    return pl.kernel(
        functools.partial(_agg_body, K, split_edges),
        out_type=jax.ShapeDtypeStruct((n_out, N_PAD, C), jnp.float32),
        mesh=mesh,
        scratch_types=[
            pltpu.VMEM((SEB,), jnp.int32),
            pltpu.VMEM((1, SEB), jnp.int32),
            pltpu.VMEM((SEB,), jnp.float32),
            pltpu.VMEM((EB, C), jnp.float32),
            pltpu.VMEM((EB, C), jnp.float32),
            pltpu.VMEM_SHARED((N_PAD, C), jnp.float32),
            pltpu.SemaphoreType.DMA,
            pltpu.SemaphoreType.DMA,
            pltpu.SemaphoreType.DMA,
            pltpu.SemaphoreType.DMA,
        ],
        compiler_params=pltpu.CompilerParams(needs_layout_passes=False),
    )


_agg_b = _make_agg(1, True)
_agg_a2 = _make_agg(2, False)
_agg_a4 = _make_agg(4, False)


def _t1_body(p0, p1, ds, w1, b1, w2, *outs):
    d = ds[...]
    h = jnp.maximum(((p0[...] + p1[...]) * d) @ w1[...] + b1[...][None, :], 0.0)
    z = jnp.dot(h, w2[...], preferred_element_type=jnp.float32) * d
    for k, o in enumerate(outs):
        o[...] = z[:, k * C:(k + 1) * C]


def _t1(p0, p1, ds, W1, b1, W2):
    kout = W2.shape[1] // C
    return pl.pallas_call(
        _t1_body,
        grid=(N // ROW_BLK,),
        in_specs=[
            pl.BlockSpec((ROW_BLK, C), lambda i: (i, 0)),
            pl.BlockSpec((ROW_BLK, C), lambda i: (i, 0)),
            pl.BlockSpec((ROW_BLK, 1), lambda i: (i, 0)),
            pl.BlockSpec(W1.shape, lambda i: (0, 0)),
            pl.BlockSpec(b1.shape, lambda i: (0,)),
            pl.BlockSpec(W2.shape, lambda i: (0, 0)),
        ],
        out_specs=[pl.BlockSpec((ROW_BLK, C), lambda i: (i, 0))] * kout,
        out_shape=[jax.ShapeDtypeStruct((N, C), jnp.float32)] * kout,
    )(p0, p1, ds, W1, b1, W2)


def _tmid_body(nchunks, *refs):
    gs = refs[:nchunks]
    ds, b, w = refs[nchunks], refs[nchunks + 1], refs[nchunks + 2]
    outs = refs[nchunks + 3:]
    d = ds[...]
    hcat = jnp.concatenate([g[...] for g in gs], axis=1)
    h = jnp.maximum(hcat * d + b[...][None, :], 0.0)
    z = jnp.dot(h, w[...], preferred_element_type=jnp.float32) * d
    for k, o in enumerate(outs):
        o[...] = z[:, k * C:(k + 1) * C]


def _tmid(gs, ds, b, W):
    kout = W.shape[1] // C
    return pl.pallas_call(
        functools.partial(_tmid_body, len(gs)),
        grid=(N // ROW_BLK,),
        in_specs=(
            [pl.BlockSpec((ROW_BLK, C), lambda i: (i, 0))] * len(gs)
            + [pl.BlockSpec((ROW_BLK, 1), lambda i: (i, 0)),
               pl.BlockSpec(b.shape, lambda i: (0,)),
               pl.BlockSpec(W.shape, lambda i: (0, 0))]
        ),
        out_specs=[pl.BlockSpec((ROW_BLK, C), lambda i: (i, 0))] * kout,
        out_shape=[jax.ShapeDtypeStruct((N, C), jnp.float32)] * kout,
    )(*gs, ds, b, W)


def _t5_body(q0, q1, ds, b5, o):
    o[...] = (q0[...] + q1[...]) * ds[...] + b5[...][None, :]


def _t5(q0, q1, ds, b5):
    return pl.pallas_call(
        _t5_body,
        grid=(N // ROW_BLK,),
        in_specs=[
            pl.BlockSpec((ROW_BLK, C), lambda i: (i, 0)),
            pl.BlockSpec((ROW_BLK, C), lambda i: (i, 0)),
            pl.BlockSpec((ROW_BLK, 1), lambda i: (i, 0)),
            pl.BlockSpec(b5.shape, lambda i: (0,)),
        ],
        out_specs=pl.BlockSpec((ROW_BLK, C), lambda i: (i, 0)),
        out_shape=jax.ShapeDtypeStruct((N, C), jnp.float32),
    )(q0, q1, ds, b5)


def kernel(x, edge_index, edge_attr, W1, b1, W2, b2, W3, b3, W4, b4, W5, b5):
    loop = jnp.arange(N, dtype=jnp.int32)
    pad = E_PAD - (E + N)
    src = jnp.concatenate([edge_index[0], loop, jnp.zeros((pad,), jnp.int32)])
    dst = jnp.concatenate([edge_index[1], loop, jnp.zeros((pad,), jnp.int32)])
    ew = jnp.concatenate([edge_attr, jnp.ones((N,), jnp.float32),
                          jnp.zeros((pad,), jnp.float32)])
    deg = jax.ops.segment_sum(ew, dst, num_segments=N)
    dis = jnp.where(deg > 0, lax.rsqrt(jnp.where(deg > 0, deg, 1.0)), 0.0)
    ds = dis[:, None]

    cut = lambda t3: [t3[k, :N] for k in range(t3.shape[0])]
    cat = lambda ts: jnp.concatenate(ts, axis=0)
    p0, p1 = cut(_agg_b(src, dst, ew, x * ds))
    z2 = _t1(p0, p1, ds, W1, b1, W2)
    g2 = cut(_agg_a4(src, dst, ew, cat(z2)))
    z3 = _tmid(g2, ds, b2, W3)
    g3 = cut(_agg_a4(src, dst, ew, cat(z3)))
    z4 = _tmid(g3, ds, b3, W4)
    g4 = cut(_agg_a2(src, dst, ew, cat(z4)))
    (z5,) = _tmid(g4, ds, b4, W5)
    q0, q1 = cut(_agg_b(src, dst, ew, z5))
    return _t5(q0, q1, ds, b5)


# speedup vs baseline: 1.0620x; 1.0620x over previous
import functools

import jax
import jax.numpy as jnp
from jax import lax
from jax.experimental import pallas as pl
from jax.experimental.pallas import tpu as pltpu
from jax.experimental.pallas import tpu_sc as plsc

N = 10000
N_PAD = 10240
E = 320000
EB = 128
E_PAD = 331776
ROW_BLK = 400
N_SLICE = N_PAD // 16
C = 128


SB = 9
SEB = SB * EB


def _agg_body(K, split_edges, src_hbm, dst_hbm, ew_hbm, h_hbm, out_hbm,
              srcbuf, dstbuf, ewbuf, rows_a, rows_b, acc,
              sem_a, sem_b, sem_sa, sem_sb):
    cid = lax.axis_index("c")
    sid = lax.axis_index("s")

    my_slice = pl.ds(sid * N_SLICE, N_SLICE)

    def zero_acc_slice():
        def zrow(r, carry):
            for j in range(C // 16):
                rows_a[r, pl.ds(j * 16, 16)] = jnp.zeros((16,), jnp.float32)
            return carry
        lax.fori_loop(0, EB, zrow, 0)
        for i in range(N_SLICE // EB):
            pltpu.sync_copy(rows_a, acc.at[pl.ds(sid * N_SLICE + i * EB, EB)])

    def edge_loop(koff, e_base, ngrp):
        koffv = jnp.full((16,), koff, jnp.int32)
        bufs = [(rows_a, sem_a, sem_sa), (rows_b, sem_b, sem_sb)]

        def grp_body(g, carry):
            e0 = e_base + g * SEB
            pltpu.sync_copy(src_hbm.at[pl.ds(e0, SEB)], srcbuf)
            pltpu.sync_copy(dst_hbm.at[pl.ds(e0, SEB)], dstbuf.at[0])
            pltpu.sync_copy(ew_hbm.at[pl.ds(e0, SEB)], ewbuf)
            for j in range(SEB // 16):
                sl = pl.ds(j * 16, 16)
                srcbuf[sl] = srcbuf[sl] + koffv

            gh = [None, None]
            sh = [None, None]

            def issue(b):
                r, s, _ = bufs[b % 2]
                if sh[b % 2] is not None:
                    sh[b % 2].wait()
                    sh[b % 2] = None
                gh[b % 2] = pltpu.async_copy(
                    h_hbm.at[srcbuf.at[pl.ds(b * EB, EB)]], r, s)

            def drain(b):
                r, _, ss = bufs[b % 2]
                gh[b % 2].wait()

                def scale_body(e4, c2):
                    for u in range(4):
                        e = e4 * 4 + u
                        ev = plsc.load_gather(
                            ewbuf, [jnp.full((16,), b * EB, jnp.int32) + e])
                        for j in range(C // 16):
                            sl = pl.ds(j * 16, 16)
                            r[e, sl] = r[e, sl] * ev
                    return c2
                lax.fori_loop(0, EB // 4, scale_body, 0)
                sh[b % 2] = pltpu.async_copy(
                    r, acc.at[dstbuf.at[0, pl.ds(b * EB, EB)]], ss, add=True)

            issue(0)
            for b in range(SB):
                if b + 1 < SB:
                    issue(b + 1)
                drain(b)
            for h in sh:
                if h is not None:
                    h.wait()
            return carry
        lax.fori_loop(0, ngrp, grp_body, 0)

    if split_edges:
        wid = cid * 16 + sid
        per_tile = E_PAD // 32
        zero_acc_slice()
        plsc.subcore_barrier()
        edge_loop(jnp.int32(0), wid * per_tile, per_tile // SEB)
        plsc.subcore_barrier()
        pltpu.sync_copy(acc.at[my_slice], out_hbm.at[cid, my_slice])
    else:
        per_tile = E_PAD // 16
        for k_local in range(K // 2):
            k = cid * (K // 2) + k_local
            zero_acc_slice()
            plsc.subcore_barrier()
            edge_loop(k * N, sid * per_tile, per_tile // SEB)
            plsc.subcore_barrier()
            pltpu.sync_copy(acc.at[my_slice], out_hbm.at[k, my_slice])


def _make_agg(K, split_edges):
    n_out = 2 if split_edges else K
    mesh = plsc.VectorSubcoreMesh(core_axis_name="c", subcore_axis_name="s")
    return pl.kernel(
        functools.partial(_agg_body, K, split_edges),
        out_type=jax.ShapeDtypeStruct((n_out, N_PAD, C), jnp.float32),
        mesh=mesh,
        scratch_types=[
            pltpu.VMEM((SEB,), jnp.int32),
            pltpu.VMEM((1, SEB), jnp.int32),
            pltpu.VMEM((SEB,), jnp.float32),
            pltpu.VMEM((EB, C), jnp.float32),
            pltpu.VMEM((EB, C), jnp.float32),
            pltpu.VMEM_SHARED((N_PAD, C), jnp.float32),
            pltpu.SemaphoreType.DMA,
            pltpu.SemaphoreType.DMA,
            pltpu.SemaphoreType.DMA,
            pltpu.SemaphoreType.DMA,
        ],
        compiler_params=pltpu.CompilerParams(needs_layout_passes=False),
    )


_agg_b = _make_agg(1, True)
_agg_a2 = _make_agg(2, False)
_agg_a4 = _make_agg(4, False)


def _deg_body(dst_hbm, ew_hbm, out_hbm, dstbuf, ewbuf, rows, acc):
    cid = lax.axis_index("c")
    sid = lax.axis_index("s")
    my_slice = pl.ds(sid * N_SLICE, N_SLICE)

    def zrow(r, carry):
        for j in range(C // 16):
            rows[r, pl.ds(j * 16, 16)] = jnp.zeros((16,), jnp.float32)
        return carry
    lax.fori_loop(0, EB, zrow, 0)
    for i in range(N_SLICE // EB):
        pltpu.sync_copy(rows, acc.at[pl.ds(sid * N_SLICE + i * EB, EB)])
    plsc.subcore_barrier()

    wid = cid * 16 + sid
    per_tile = E_PAD // 32

    def grp_body(g, carry):
        e0 = wid * per_tile + g * SEB
        pltpu.sync_copy(dst_hbm.at[pl.ds(e0, SEB)], dstbuf.at[0])
        pltpu.sync_copy(ew_hbm.at[pl.ds(e0, SEB)], ewbuf)
        for b in range(SB):
            def fill(e4, c2):
                for u in range(4):
                    e = e4 * 4 + u
                    ev = plsc.load_gather(
                        ewbuf, [jnp.full((16,), b * EB, jnp.int32) + e])
                    for j in range(C // 16):
                        rows[e, pl.ds(j * 16, 16)] = ev
                return c2
            lax.fori_loop(0, EB // 4, fill, 0)
            pltpu.sync_copy(
                rows, acc.at[dstbuf.at[0, pl.ds(b * EB, EB)]], add=True)
        return carry
    lax.fori_loop(0, per_tile // SEB, grp_body, 0)
    plsc.subcore_barrier()
    pltpu.sync_copy(acc.at[my_slice], out_hbm.at[cid, my_slice])


_deg = pl.kernel(
    _deg_body,
    out_type=jax.ShapeDtypeStruct((2, N_PAD, C), jnp.float32),
    mesh=plsc.VectorSubcoreMesh(core_axis_name="c", subcore_axis_name="s"),
    scratch_types=[
        pltpu.VMEM((1, SEB), jnp.int32),
        pltpu.VMEM((SEB,), jnp.float32),
        pltpu.VMEM((EB, C), jnp.float32),
        pltpu.VMEM_SHARED((N_PAD, C), jnp.float32),
    ],
    compiler_params=pltpu.CompilerParams(needs_layout_passes=False),
)


def _t1_body(p0, p1, ds, w1, b1, w2, *outs):
    d = ds[...]
    h = jnp.maximum(((p0[...] + p1[...]) * d) @ w1[...] + b1[...][None, :], 0.0)
    z = jnp.dot(h, w2[...], preferred_element_type=jnp.float32) * d
    for k, o in enumerate(outs):
        o[...] = z[:, k * C:(k + 1) * C]


def _t1(p0, p1, ds, W1, b1, W2):
    kout = W2.shape[1] // C
    return pl.pallas_call(
        _t1_body,
        grid=(N // ROW_BLK,),
        in_specs=[
            pl.BlockSpec((ROW_BLK, C), lambda i: (i, 0)),
            pl.BlockSpec((ROW_BLK, C), lambda i: (i, 0)),
            pl.BlockSpec((ROW_BLK, 1), lambda i: (i, 0)),
            pl.BlockSpec(W1.shape, lambda i: (0, 0)),
            pl.BlockSpec(b1.shape, lambda i: (0,)),
            pl.BlockSpec(W2.shape, lambda i: (0, 0)),
        ],
        out_specs=[pl.BlockSpec((ROW_BLK, C), lambda i: (i, 0))] * kout,
        out_shape=[jax.ShapeDtypeStruct((N, C), jnp.float32)] * kout,
    )(p0, p1, ds, W1, b1, W2)


def _tmid_body(nchunks, *refs):
    gs = refs[:nchunks]
    ds, b, w = refs[nchunks], refs[nchunks + 1], refs[nchunks + 2]
    outs = refs[nchunks + 3:]
    d = ds[...]
    hcat = jnp.concatenate([g[...] for g in gs], axis=1)
    h = jnp.maximum(hcat * d + b[...][None, :], 0.0)
    z = jnp.dot(h, w[...], preferred_element_type=jnp.float32) * d
    for k, o in enumerate(outs):
        o[...] = z[:, k * C:(k + 1) * C]


def _tmid(gs, ds, b, W):
    kout = W.shape[1] // C
    return pl.pallas_call(
        functools.partial(_tmid_body, len(gs)),
        grid=(N // ROW_BLK,),
        in_specs=(
            [pl.BlockSpec((ROW_BLK, C), lambda i: (i, 0))] * len(gs)
            + [pl.BlockSpec((ROW_BLK, 1), lambda i: (i, 0)),
               pl.BlockSpec(b.shape, lambda i: (0,)),
               pl.BlockSpec(W.shape, lambda i: (0, 0))]
        ),
        out_specs=[pl.BlockSpec((ROW_BLK, C), lambda i: (i, 0))] * kout,
        out_shape=[jax.ShapeDtypeStruct((N, C), jnp.float32)] * kout,
    )(*gs, ds, b, W)


def _t5_body(q0, q1, ds, b5, o):
    o[...] = (q0[...] + q1[...]) * ds[...] + b5[...][None, :]


def _t5(q0, q1, ds, b5):
    return pl.pallas_call(
        _t5_body,
        grid=(N // ROW_BLK,),
        in_specs=[
            pl.BlockSpec((ROW_BLK, C), lambda i: (i, 0)),
            pl.BlockSpec((ROW_BLK, C), lambda i: (i, 0)),
            pl.BlockSpec((ROW_BLK, 1), lambda i: (i, 0)),
            pl.BlockSpec(b5.shape, lambda i: (0,)),
        ],
        out_specs=pl.BlockSpec((ROW_BLK, C), lambda i: (i, 0)),
        out_shape=jax.ShapeDtypeStruct((N, C), jnp.float32),
    )(q0, q1, ds, b5)


def kernel(x, edge_index, edge_attr, W1, b1, W2, b2, W3, b3, W4, b4, W5, b5):
    loop = jnp.arange(N, dtype=jnp.int32)
    pad = E_PAD - (E + N)
    src = jnp.concatenate([edge_index[0], loop, jnp.zeros((pad,), jnp.int32)])
    dst = jnp.concatenate([edge_index[1], loop, jnp.zeros((pad,), jnp.int32)])
    ew = jnp.concatenate([edge_attr, jnp.ones((N,), jnp.float32),
                          jnp.zeros((pad,), jnp.float32)])
    dpart = _deg(dst, ew)
    deg = dpart[0, :N, 0] + dpart[1, :N, 0]
    dis = jnp.where(deg > 0, lax.rsqrt(jnp.where(deg > 0, deg, 1.0)), 0.0)
    ds = dis[:, None]

    cut = lambda t3: [t3[k, :N] for k in range(t3.shape[0])]
    cat = lambda ts: jnp.concatenate(ts, axis=0)
    p0, p1 = cut(_agg_b(src, dst, ew, x * ds))
    z2 = _t1(p0, p1, ds, W1, b1, W2)
    g2 = cut(_agg_a4(src, dst, ew, cat(z2)))
    z3 = _tmid(g2, ds, b2, W3)
    g3 = cut(_agg_a4(src, dst, ew, cat(z3)))
    z4 = _tmid(g3, ds, b3, W4)
    g4 = cut(_agg_a2(src, dst, ew, cat(z4)))
    (z5,) = _tmid(g4, ds, b4, W5)
    q0, q1 = cut(_agg_b(src, dst, ew, z5))
    return _t5(q0, q1, ds, b5)
